# Initial kernel scaffold; baseline (speedup 1.0000x reference)
import jax
import jax.numpy as jnp
from jax.experimental import pallas as pl


def kernel(inputs, word_embedding, position_embedding):
    raise NotImplementedError("write your pallas kernel here")


# speedup vs baseline: 1.0501x; 1.0501x over previous
import functools

import jax
import jax.numpy as jnp
from jax import lax
from jax.experimental import pallas as pl
from jax.experimental.pallas import tpu as pltpu
from jax.experimental.pallas import tpu_sc as plsc

D = 128
SEQ = 2048
BATCH = 4
B_TOTAL = BATCH * SEQ
NC, NS, L = 2, 16, 16
NW = NC * NS
BPW = B_TOTAL // NW
GCH = 128
NG = BPW // GCH


def _emb_body(idx_hbm, word_hbm, pos_hbm, out_hbm, idx_v, rows_v, pos_v, sem):
    wid = lax.axis_index("s") * NC + lax.axis_index("c")
    base = wid * BPW
    pos_base = lax.rem(base, SEQ)

    for g in range(NG):
        pltpu.sync_copy(idx_hbm.at[pl.ds(base + g * GCH, GCH)], idx_v.at[g])

    copies = [
        pltpu.async_copy(word_hbm.at[idx_v.at[g]],
                         rows_v.at[pl.ds(g * GCH, GCH)], sem)
        for g in range(NG)
    ]
    pltpu.sync_copy(pos_hbm.at[pl.ds(pos_base, BPW)], pos_v)
    for c in copies:
        c.wait()

    def add_row(j, _):
        for k in range(D // L):
            sl = pl.ds(k * L, L)
            rows_v[j, sl] = rows_v[j, sl] + pos_v[j, sl]
        return 0

    lax.fori_loop(0, BPW, add_row, 0)

    pltpu.sync_copy(rows_v, out_hbm.at[pl.ds(base, BPW)])


@jax.jit
def kernel(inputs, word_embedding, position_embedding):
    idx = inputs.astype(jnp.int32).reshape(B_TOTAL)
    mesh = plsc.VectorSubcoreMesh(core_axis_name="c", subcore_axis_name="s")
    out = pl.kernel(
        _emb_body,
        mesh=mesh,
        out_type=jax.ShapeDtypeStruct((B_TOTAL, D), jnp.float32),
        scratch_types=[
            pltpu.VMEM((NG, GCH), jnp.int32),
            pltpu.VMEM((BPW, D), jnp.float32),
            pltpu.VMEM((BPW, D), jnp.float32),
            pltpu.SemaphoreType.DMA,
        ],
    )(idx, word_embedding, position_embedding)
    return out.reshape(BATCH, SEQ, D)


# speedup vs baseline: 1.1044x; 1.0517x over previous
import functools

import jax
import jax.numpy as jnp
from jax import lax
from jax.experimental import pallas as pl
from jax.experimental.pallas import tpu as pltpu
from jax.experimental.pallas import tpu_sc as plsc

D = 128
SEQ = 2048
BATCH = 4
B_TOTAL = BATCH * SEQ
NC, NS, L = 2, 16, 16
NW = NC * NS
BPW = B_TOTAL // NW
NCH = 4
CH = BPW // NCH


def _emb_body(idx_hbm, word_hbm, pos_hbm, out_hbm, idx_v, rows_v, pos_v,
              isem, gsem, psem, osem):
    wid = lax.axis_index("s") * NC + lax.axis_index("c")
    base = wid * BPW
    pos_base = lax.rem(base, SEQ)

    idxcp = [
        pltpu.async_copy(idx_hbm.at[pl.ds(base + g * CH, CH)],
                         idx_v.at[g], isem.at[g])
        for g in range(NCH)
    ]
    poscp = [
        pltpu.async_copy(pos_hbm.at[pl.ds(pos_base + g * CH, CH)],
                         pos_v.at[pl.ds(g * CH, CH)], psem.at[g])
        for g in range(NCH)
    ]

    gathers = []
    for g in range(NCH):
        idxcp[g].wait()
        gathers.append(
            pltpu.async_copy(word_hbm.at[idx_v.at[g]],
                             rows_v.at[pl.ds(g * CH, CH)], gsem.at[g]))

    outs = []
    for g in range(NCH):
        gathers[g].wait()
        poscp[g].wait()

        def add_row(j, _, g=g):
            row = g * CH + j
            for k in range(D // L):
                sl = pl.ds(k * L, L)
                plsc.addupdate(rows_v.at[row, sl], pos_v[row, sl])
            return 0

        lax.fori_loop(0, CH, add_row, 0)
        outs.append(
            pltpu.async_copy(rows_v.at[pl.ds(g * CH, CH)],
                             out_hbm.at[pl.ds(base + g * CH, CH)],
                             osem.at[g]))
    for o in outs:
        o.wait()


@jax.jit
def kernel(inputs, word_embedding, position_embedding):
    idx = inputs.astype(jnp.int32).reshape(B_TOTAL)
    mesh = plsc.VectorSubcoreMesh(core_axis_name="c", subcore_axis_name="s")
    out = pl.kernel(
        _emb_body,
        mesh=mesh,
        out_type=jax.ShapeDtypeStruct((B_TOTAL, D), jnp.float32),
        scratch_types=[
            pltpu.VMEM((NCH, CH), jnp.int32),
            pltpu.VMEM((BPW, D), jnp.float32),
            pltpu.VMEM((BPW, D), jnp.float32),
            pltpu.SemaphoreType.DMA((NCH,)),
            pltpu.SemaphoreType.DMA((NCH,)),
            pltpu.SemaphoreType.DMA((NCH,)),
            pltpu.SemaphoreType.DMA((NCH,)),
        ],
    )(idx, word_embedding, position_embedding)
    return out.reshape(BATCH, SEQ, D)


# speedup vs baseline: 1.1750x; 1.0639x over previous
import functools

import jax
import jax.numpy as jnp
from jax import lax
from jax.experimental import pallas as pl
from jax.experimental.pallas import tpu as pltpu
from jax.experimental.pallas import tpu_sc as plsc

D = 128
SEQ = 2048
BATCH = 4
B_TOTAL = BATCH * SEQ
NC, NS, L = 2, 16, 16
NW = NC * NS
BPW = B_TOTAL // NW
SCH = SEQ // NW


def _emb_body(idx_hbm, word_hbm, pos_hbm, out_hbm, idx_v, rows_v, pos_v,
              isem, gsem, psem, osem):
    wid = lax.axis_index("s") * NC + lax.axis_index("c")
    s0 = wid * SCH

    idxcp = [
        pltpu.async_copy(idx_hbm.at[pl.ds(b * SEQ + s0, SCH)],
                         idx_v.at[b], isem.at[b])
        for b in range(BATCH)
    ]
    poscp = pltpu.async_copy(pos_hbm.at[pl.ds(s0, SCH)], pos_v, psem)

    gathers = []
    for b in range(BATCH):
        idxcp[b].wait()
        gathers.append(
            pltpu.async_copy(word_hbm.at[idx_v.at[b]],
                             rows_v.at[pl.ds(b * SCH, SCH)], gsem.at[b]))

    poscp.wait()
    outs = []
    for b in range(BATCH):
        gathers[b].wait()

        def add_row(j, _, b=b):
            row = b * SCH + j
            for k in range(D // L):
                sl = pl.ds(k * L, L)
                plsc.addupdate(rows_v.at[row, sl], pos_v[j, sl])
            return 0

        lax.fori_loop(0, SCH, add_row, 0)
        outs.append(
            pltpu.async_copy(rows_v.at[pl.ds(b * SCH, SCH)],
                             out_hbm.at[pl.ds(b * SEQ + s0, SCH)],
                             osem.at[b]))
    for o in outs:
        o.wait()


@jax.jit
def kernel(inputs, word_embedding, position_embedding):
    idx = inputs.astype(jnp.int32).reshape(B_TOTAL)
    mesh = plsc.VectorSubcoreMesh(core_axis_name="c", subcore_axis_name="s")
    out = pl.kernel(
        _emb_body,
        mesh=mesh,
        out_type=jax.ShapeDtypeStruct((B_TOTAL, D), jnp.float32),
        scratch_types=[
            pltpu.VMEM((BATCH, SCH), jnp.int32),
            pltpu.VMEM((BPW, D), jnp.float32),
            pltpu.VMEM((SCH, D), jnp.float32),
            pltpu.SemaphoreType.DMA((BATCH,)),
            pltpu.SemaphoreType.DMA((BATCH,)),
            pltpu.SemaphoreType.DMA,
            pltpu.SemaphoreType.DMA((BATCH,)),
        ],
    )(idx, word_embedding, position_embedding)
    return out.reshape(BATCH, SEQ, D)
